# Initial kernel scaffold; baseline (speedup 1.0000x reference)
import jax
import jax.numpy as jnp
from jax.experimental import pallas as pl


def kernel(x, wb1, b1, wb2, b2, wf1, bf1, wf2, bf2, wf3, bf3):
    raise NotImplementedError("write your pallas kernel here")


# speedup vs baseline: 1.2817x; 1.2817x over previous
import functools

import jax
import jax.numpy as jnp
import numpy as np
from jax.experimental import pallas as pl
from jax.experimental.pallas import tpu as pltpu

NB = 32
SLOTS = NB + 1
L = NB * 8
L2 = L - 2


def _lenet_body(a_ref, b_ref, w0_ref, w1_ref, w2_ref, b1_ref,
                v0_ref, v1_ref, v2_ref, b2_ref, s_ref,
                f01_ref, f23_ref, f4_ref, bf1_ref,
                wf2_ref, bf2_ref, wf3_ref, bf3_ref, o_ref):
    f32, bf16 = jnp.float32, jnp.bfloat16
    dot = functools.partial(jnp.dot, preferred_element_type=f32)
    A = a_ref[0]
    B = b_ref[0]

    def pool_relu(y, bias):
        m = jnp.maximum(y[:, :256], y[:, 256:])
        m = jnp.maximum(m[:, :128], m[:, 128:]) + bias
        return jnp.maximum(m, 0.0).astype(bf16)

    y01 = (dot(A[0:L], w0_ref[...]) + dot(B[0:L], w1_ref[...])
           + dot(A[1:1 + L], w2_ref[...]))
    ze = pool_relu(y01, b1_ref[...])
    y23 = (dot(B[0:L], w0_ref[...]) + dot(A[1:1 + L], w1_ref[...])
           + dot(B[1:1 + L], w2_ref[...]))
    zo = pool_relu(y23, b1_ref[...])
    C = jnp.concatenate([ze, zo], axis=1)

    u = (dot(C[0:L2], v0_ref[...]) + dot(C[1:1 + L2], v1_ref[...])
         + dot(C[2:2 + L2], v2_ref[...]))
    p2 = pool_relu(u, b2_ref[...])

    sg = dot(s_ref[...], p2).astype(bf16)
    h1 = (dot(jnp.concatenate([sg[0:NB], sg[NB:2 * NB]], axis=1), f01_ref[...])
          + dot(jnp.concatenate([sg[2 * NB:3 * NB], sg[3 * NB:4 * NB]], axis=1),
                f23_ref[...])
          + dot(sg[4 * NB:5 * NB], f4_ref[...]))
    h1 = jnp.maximum(h1 + bf1_ref[...], 0.0).astype(bf16)
    h2 = jnp.maximum(dot(h1, wf2_ref[...]) + bf2_ref[...], 0.0).astype(bf16)
    o_ref[0] = dot(h2, wf3_ref[...]) + bf3_ref[...]


def _pack_weights(wb1, wb2, wf1):
    bf16 = jnp.bfloat16
    z = jnp.zeros((128, 256), bf16)

    def pair(wa, wb, wc, wd):
        return jnp.concatenate(
            [jnp.concatenate([wa, wb], axis=0),
             jnp.concatenate([wc, wd], axis=0)], axis=1)

    w0 = pair(wb1[0], wb1[1], z, wb1[0])
    w1 = pair(wb1[2], wb1[3], wb1[1], wb1[2])
    w2 = pair(wb1[4], z, wb1[3], wb1[4])
    v0 = pair(wb2[0], wb2[1], z, wb2[0])
    v1 = pair(wb2[2], wb2[3], wb2[1], wb2[2])
    v2 = pair(wb2[4], z, wb2[3], wb2[4])
    f01 = jnp.concatenate([wf1[0], wf1[1]], axis=0)
    f23 = jnp.concatenate([wf1[2], wf1[3]], axis=0)
    f4 = wf1[4]
    g = np.arange(5 * NB)
    cols = 8 * (g % NB) + g // NB
    s = np.zeros((5 * NB, L2), np.float32)
    s[g, cols] = 1.0
    return w0, w1, w2, v0, v1, v2, jnp.asarray(s, bf16), f01, f23, f4


def kernel(x, wb1, b1, wb2, b2, wf1, bf1, wf2, bf2, wf3, bf3):
    Bt, Cc, H, W = x.shape
    assert (Cc, H, W) == (3, 32, 32)
    nsteps = max(1, -(-Bt // NB))
    Bp = nsteps * NB

    xt = jnp.transpose(x, (0, 2, 3, 1))
    xt = jnp.pad(xt, ((0, Bp - Bt), (0, 0), (0, 0), (0, 1)))
    rows = xt.reshape(Bp, 32, 128).astype(jnp.bfloat16)
    r = rows.reshape(nsteps, NB, 8, 4, 128)
    A = r[:, :, :, 0:2, :].reshape(nsteps, NB, 8, 256)
    B = r[:, :, :, 2:4, :].reshape(nsteps, NB, 8, 256)
    A = jnp.pad(A, ((0, 0), (0, 1), (0, 0), (0, 0))).reshape(nsteps, SLOTS * 8, 256)
    B = jnp.pad(B, ((0, 0), (0, 1), (0, 0), (0, 0))).reshape(nsteps, SLOTS * 8, 256)

    w0, w1, w2, v0, v1, v2, s, f01, f23, f4 = _pack_weights(wb1, wb2, wf1)

    c2 = lambda i: (0, 0)
    full = lambda shape: pl.BlockSpec(shape, c2)
    out = pl.pallas_call(
        _lenet_body,
        out_shape=jax.ShapeDtypeStruct((nsteps, NB, 128), jnp.float32),
        grid=(nsteps,),
        in_specs=[
            pl.BlockSpec((1, SLOTS * 8, 256), lambda i: (i, 0, 0)),
            pl.BlockSpec((1, SLOTS * 8, 256), lambda i: (i, 0, 0)),
            full((256, 512)), full((256, 512)), full((256, 512)),
            full((1, 128)),
            full((256, 512)), full((256, 512)), full((256, 512)),
            full((1, 128)),
            full((5 * NB, L2)),
            full((256, 128)), full((256, 128)), full((128, 128)),
            full((1, 128)),
            full((128, 128)), full((1, 128)),
            full((128, 128)), full((1, 128)),
        ],
        out_specs=pl.BlockSpec((1, NB, 128), lambda i: (i, 0, 0)),
        compiler_params=pltpu.CompilerParams(
            dimension_semantics=("parallel",),
            vmem_limit_bytes=64 * 1024 * 1024),
    )(A, B, w0, w1, w2, b1, v0, v1, v2, b2, s,
      f01, f23, f4, bf1, wf2, bf2, wf3, bf3)
    return out.reshape(Bp, 128)[:Bt, :10]


# speedup vs baseline: 1.8818x; 1.4682x over previous
import functools

import jax
import jax.numpy as jnp
import numpy as np
from jax.experimental import pallas as pl
from jax.experimental.pallas import tpu as pltpu

NB = 32
SLOTS = NB + 1
L = NB * 8
L2 = L - 2


def _lenet_body(x_ref, g0_ref, g1_ref, h0_ref, h1_ref, b1_ref,
                v0_ref, v1_ref, v2_ref, b2_ref, s_ref,
                f01_ref, f23_ref, f4_ref, bf1_ref,
                wf2_ref, bf2_ref, wf3_ref, bf3_ref, o_ref):
    f32, bf16 = jnp.float32, jnp.bfloat16
    dot = functools.partial(jnp.dot, preferred_element_type=f32)

    xb = x_ref[0]
    X = jnp.concatenate(
        [xb[:, c].reshape(NB * 8, 128).astype(bf16) for c in range(3)],
        axis=1)
    X = jnp.concatenate([X, jnp.zeros((8, 384), bf16)], axis=0)

    def pool_relu(y, bias):
        m = jnp.maximum(y[:, :256], y[:, 256:])
        m = jnp.maximum(m[:, :128], m[:, 128:]) + bias
        return jnp.maximum(m, 0.0).astype(bf16)

    y01 = dot(X[0:L], g0_ref[...]) + dot(X[1:1 + L], g1_ref[...])
    ze = pool_relu(y01, b1_ref[...])
    y23 = dot(X[0:L], h0_ref[...]) + dot(X[1:1 + L], h1_ref[...])
    zo = pool_relu(y23, b1_ref[...])
    C = jnp.concatenate([ze, zo], axis=1)

    u = (dot(C[0:L2], v0_ref[...]) + dot(C[1:1 + L2], v1_ref[...])
         + dot(C[2:2 + L2], v2_ref[...]))
    p2 = pool_relu(u, b2_ref[...])

    sg = dot(s_ref[...], p2).astype(bf16)
    h1 = (dot(jnp.concatenate([sg[0:NB], sg[NB:2 * NB]], axis=1), f01_ref[...])
          + dot(jnp.concatenate([sg[2 * NB:3 * NB], sg[3 * NB:4 * NB]], axis=1),
                f23_ref[...])
          + dot(sg[4 * NB:5 * NB], f4_ref[...]))
    h1 = jnp.maximum(h1 + bf1_ref[...], 0.0).astype(bf16)
    h2 = jnp.maximum(dot(h1, wf2_ref[...]) + bf2_ref[...], 0.0).astype(bf16)
    o_ref[0] = dot(h2, wf3_ref[...]) + bf3_ref[...]


def _pack_weights(wb1, wb2, wf1):
    bf16 = jnp.bfloat16
    z = jnp.zeros((128, 256), bf16)

    def pair(wa, wb, wc, wd):
        return jnp.concatenate(
            [jnp.concatenate([wa, wb], axis=0),
             jnp.concatenate([wc, wd], axis=0)], axis=1)

    w0 = pair(wb1[0], wb1[1], z, wb1[0])
    w1 = pair(wb1[2], wb1[3], wb1[1], wb1[2])
    w2 = pair(wb1[4], z, wb1[3], wb1[4])
    c, r4, w = np.meshgrid(np.arange(3), np.arange(4), np.arange(32),
                           indexing="ij")
    q = jnp.asarray((r4 * 128 + w * 4 + c).reshape(384))
    z512 = jnp.zeros((256, 512), bf16)
    vs = lambda a, b: jnp.concatenate([a, b], axis=0)
    g0 = vs(w0, w1)[q]
    g1 = vs(w2, z512)[q]
    h0 = vs(z512, w0)[q]
    h1 = vs(w1, w2)[q]
    v0 = pair(wb2[0], wb2[1], z, wb2[0])
    v1 = pair(wb2[2], wb2[3], wb2[1], wb2[2])
    v2 = pair(wb2[4], z, wb2[3], wb2[4])
    f01 = jnp.concatenate([wf1[0], wf1[1]], axis=0)
    f23 = jnp.concatenate([wf1[2], wf1[3]], axis=0)
    f4 = wf1[4]
    g = np.arange(5 * NB)
    cols = 8 * (g % NB) + g // NB
    s = np.zeros((5 * NB, L2), np.float32)
    s[g, cols] = 1.0
    return g0, g1, h0, h1, v0, v1, v2, jnp.asarray(s, bf16), f01, f23, f4


def kernel(x, wb1, b1, wb2, b2, wf1, bf1, wf2, bf2, wf3, bf3):
    Bt, Cc, H, W = x.shape
    assert (Cc, H, W) == (3, 32, 32)
    nsteps = max(1, -(-Bt // NB))
    Bp = nsteps * NB
    if Bp != Bt:
        x = jnp.pad(x, ((0, Bp - Bt), (0, 0), (0, 0), (0, 0)))
    xv = x.reshape(nsteps, NB, 3, 8, 128)

    g0, g1, h0, h1, v0, v1, v2, s, f01, f23, f4 = _pack_weights(wb1, wb2, wf1)

    c2 = lambda i: (0, 0)
    full = lambda shape: pl.BlockSpec(shape, c2)
    out = pl.pallas_call(
        _lenet_body,
        out_shape=jax.ShapeDtypeStruct((nsteps, NB, 128), jnp.float32),
        grid=(nsteps,),
        in_specs=[
            pl.BlockSpec((1, NB, 3, 8, 128), lambda i: (i, 0, 0, 0, 0)),
            full((384, 512)), full((384, 512)),
            full((384, 512)), full((384, 512)),
            full((1, 128)),
            full((256, 512)), full((256, 512)), full((256, 512)),
            full((1, 128)),
            full((5 * NB, L2)),
            full((256, 128)), full((256, 128)), full((128, 128)),
            full((1, 128)),
            full((128, 128)), full((1, 128)),
            full((128, 128)), full((1, 128)),
        ],
        out_specs=pl.BlockSpec((1, NB, 128), lambda i: (i, 0, 0)),
        compiler_params=pltpu.CompilerParams(
            dimension_semantics=("parallel",),
            vmem_limit_bytes=64 * 1024 * 1024),
    )(xv, g0, g1, h0, h1, b1, v0, v1, v2, b2, s,
      f01, f23, f4, bf1, wf2, bf2, wf3, bf3)
    return out.reshape(Bp, 128)[:Bt, :10]


# speedup vs baseline: 2.7372x; 1.4545x over previous
import functools

import jax
import jax.numpy as jnp
import numpy as np
from jax.experimental import pallas as pl
from jax.experimental.pallas import tpu as pltpu

NB = 64
SLOTS = NB + 1
L = NB * 8
L2 = L - 2


def _lenet_body(x_ref, g0_ref, g1_ref, h0_ref, h1_ref, b1_ref,
                v0_ref, v1_ref, v2_ref, b2_ref, s_ref,
                f01_ref, f23_ref, f4_ref, bf1_ref,
                wf2_ref, bf2_ref, wf3_ref, bf3_ref, o_ref):
    f32, bf16 = jnp.float32, jnp.bfloat16
    dot = functools.partial(jnp.dot, preferred_element_type=f32)

    xb = x_ref[0]
    X = jnp.concatenate(
        [xb[:, c].reshape(NB * 8, 128).astype(bf16) for c in range(3)],
        axis=1)
    X = jnp.concatenate([X, jnp.zeros((8, 384), bf16)], axis=0)

    def pool_relu(y, bias):
        m = jnp.maximum(y[:, :256], y[:, 256:])
        m = jnp.maximum(m[:, :128], m[:, 128:]) + bias
        return jnp.maximum(m, 0.0).astype(bf16)

    y01 = dot(X[0:L], g0_ref[...]) + dot(X[1:1 + L], g1_ref[...])
    ze = pool_relu(y01, b1_ref[...])
    y23 = dot(X[0:L], h0_ref[...]) + dot(X[1:1 + L], h1_ref[...])
    zo = pool_relu(y23, b1_ref[...])
    C = jnp.concatenate([ze, zo], axis=1)

    u = (dot(C[0:L2], v0_ref[...]) + dot(C[1:1 + L2], v1_ref[...])
         + dot(C[2:2 + L2], v2_ref[...]))
    p2 = pool_relu(u, b2_ref[...])

    sg = dot(s_ref[...], p2).astype(bf16)
    h1 = (dot(jnp.concatenate([sg[0:NB], sg[NB:2 * NB]], axis=1), f01_ref[...])
          + dot(jnp.concatenate([sg[2 * NB:3 * NB], sg[3 * NB:4 * NB]], axis=1),
                f23_ref[...])
          + dot(sg[4 * NB:5 * NB], f4_ref[...]))
    h1 = jnp.maximum(h1 + bf1_ref[...], 0.0).astype(bf16)
    h2 = jnp.maximum(dot(h1, wf2_ref[...]) + bf2_ref[...], 0.0).astype(bf16)
    o_ref[0] = dot(h2, wf3_ref[...]) + bf3_ref[...]


def _pack_weights(wb1, wb2, wf1):
    bf16 = jnp.bfloat16
    z = jnp.zeros((128, 256), bf16)

    def pair(wa, wb, wc, wd):
        return jnp.concatenate(
            [jnp.concatenate([wa, wb], axis=0),
             jnp.concatenate([wc, wd], axis=0)], axis=1)

    w0 = pair(wb1[0], wb1[1], z, wb1[0])
    w1 = pair(wb1[2], wb1[3], wb1[1], wb1[2])
    w2 = pair(wb1[4], z, wb1[3], wb1[4])
    c, r4, w = np.meshgrid(np.arange(3), np.arange(4), np.arange(32),
                           indexing="ij")
    q = jnp.asarray((r4 * 128 + w * 4 + c).reshape(384))
    z512 = jnp.zeros((256, 512), bf16)
    vs = lambda a, b: jnp.concatenate([a, b], axis=0)
    g0 = vs(w0, w1)[q]
    g1 = vs(w2, z512)[q]
    h0 = vs(z512, w0)[q]
    h1 = vs(w1, w2)[q]
    v0 = pair(wb2[0], wb2[1], z, wb2[0])
    v1 = pair(wb2[2], wb2[3], wb2[1], wb2[2])
    v2 = pair(wb2[4], z, wb2[3], wb2[4])
    f01 = jnp.concatenate([wf1[0], wf1[1]], axis=0)
    f23 = jnp.concatenate([wf1[2], wf1[3]], axis=0)
    f4 = wf1[4]
    g = np.arange(5 * NB)
    cols = 8 * (g % NB) + g // NB
    s = np.zeros((5 * NB, L2), np.float32)
    s[g, cols] = 1.0
    return g0, g1, h0, h1, v0, v1, v2, jnp.asarray(s, bf16), f01, f23, f4


def kernel(x, wb1, b1, wb2, b2, wf1, bf1, wf2, bf2, wf3, bf3):
    Bt, Cc, H, W = x.shape
    assert (Cc, H, W) == (3, 32, 32)
    nsteps = max(1, -(-Bt // NB))
    Bp = nsteps * NB
    if Bp != Bt:
        x = jnp.pad(x, ((0, Bp - Bt), (0, 0), (0, 0), (0, 0)))
    xv = x.reshape(nsteps, NB, 3, 8, 128)

    g0, g1, h0, h1, v0, v1, v2, s, f01, f23, f4 = _pack_weights(wb1, wb2, wf1)

    c2 = lambda i: (0, 0)
    full = lambda shape: pl.BlockSpec(shape, c2)
    out = pl.pallas_call(
        _lenet_body,
        out_shape=jax.ShapeDtypeStruct((nsteps, NB, 128), jnp.float32),
        grid=(nsteps,),
        in_specs=[
            pl.BlockSpec((1, NB, 3, 8, 128), lambda i: (i, 0, 0, 0, 0)),
            full((384, 512)), full((384, 512)),
            full((384, 512)), full((384, 512)),
            full((1, 128)),
            full((256, 512)), full((256, 512)), full((256, 512)),
            full((1, 128)),
            full((5 * NB, L2)),
            full((256, 128)), full((256, 128)), full((128, 128)),
            full((1, 128)),
            full((128, 128)), full((1, 128)),
            full((128, 128)), full((1, 128)),
        ],
        out_specs=pl.BlockSpec((1, NB, 128), lambda i: (i, 0, 0)),
        compiler_params=pltpu.CompilerParams(
            dimension_semantics=("parallel",),
            vmem_limit_bytes=64 * 1024 * 1024),
    )(xv, g0, g1, h0, h1, b1, v0, v1, v2, b2, s,
      f01, f23, f4, bf1, wf2, bf2, wf3, bf3)
    return out.reshape(Bp, 128)[:Bt, :10]


# speedup vs baseline: 3.0548x; 1.1160x over previous
import functools

import jax
import jax.numpy as jnp
import numpy as np
from jax.experimental import pallas as pl
from jax.experimental.pallas import tpu as pltpu

NB = 128
SLOTS = NB + 1
L = NB * 8
L2 = L - 2


def _lenet_body(x_ref, g0_ref, g1_ref, h0_ref, h1_ref, b1_ref,
                v0_ref, v1_ref, v2_ref, b2_ref, s_ref,
                f01_ref, f23_ref, f4_ref, bf1_ref,
                wf2_ref, bf2_ref, wf3_ref, bf3_ref, o_ref):
    f32, bf16 = jnp.float32, jnp.bfloat16
    dot = functools.partial(jnp.dot, preferred_element_type=f32)

    xb = x_ref[0]
    X = jnp.concatenate(
        [xb[:, c].reshape(NB * 8, 128).astype(bf16) for c in range(3)],
        axis=1)
    X = jnp.concatenate([X, jnp.zeros((8, 384), bf16)], axis=0)

    def pool_relu(y, bias):
        m = jnp.maximum(y[:, :256], y[:, 256:])
        m = jnp.maximum(m[:, :128], m[:, 128:]) + bias
        return jnp.maximum(m, 0.0).astype(bf16)

    y01 = dot(X[0:L], g0_ref[...]) + dot(X[1:1 + L], g1_ref[...])
    ze = pool_relu(y01, b1_ref[...])
    y23 = dot(X[0:L], h0_ref[...]) + dot(X[1:1 + L], h1_ref[...])
    zo = pool_relu(y23, b1_ref[...])
    C = jnp.concatenate([ze, zo], axis=1)

    u = (dot(C[0:L2], v0_ref[...]) + dot(C[1:1 + L2], v1_ref[...])
         + dot(C[2:2 + L2], v2_ref[...]))
    p2 = pool_relu(u, b2_ref[...])

    sg = dot(s_ref[...], p2).astype(bf16)
    h1 = (dot(jnp.concatenate([sg[0:NB], sg[NB:2 * NB]], axis=1), f01_ref[...])
          + dot(jnp.concatenate([sg[2 * NB:3 * NB], sg[3 * NB:4 * NB]], axis=1),
                f23_ref[...])
          + dot(sg[4 * NB:5 * NB], f4_ref[...]))
    h1 = jnp.maximum(h1 + bf1_ref[...], 0.0).astype(bf16)
    h2 = jnp.maximum(dot(h1, wf2_ref[...]) + bf2_ref[...], 0.0).astype(bf16)
    o_ref[0] = dot(h2, wf3_ref[...]) + bf3_ref[...]


def _pack_weights(wb1, wb2, wf1):
    bf16 = jnp.bfloat16
    z = jnp.zeros((128, 256), bf16)

    def pair(wa, wb, wc, wd):
        return jnp.concatenate(
            [jnp.concatenate([wa, wb], axis=0),
             jnp.concatenate([wc, wd], axis=0)], axis=1)

    w0 = pair(wb1[0], wb1[1], z, wb1[0])
    w1 = pair(wb1[2], wb1[3], wb1[1], wb1[2])
    w2 = pair(wb1[4], z, wb1[3], wb1[4])
    c, r4, w = np.meshgrid(np.arange(3), np.arange(4), np.arange(32),
                           indexing="ij")
    q = jnp.asarray((r4 * 128 + w * 4 + c).reshape(384))
    z512 = jnp.zeros((256, 512), bf16)
    vs = lambda a, b: jnp.concatenate([a, b], axis=0)
    g0 = vs(w0, w1)[q]
    g1 = vs(w2, z512)[q]
    h0 = vs(z512, w0)[q]
    h1 = vs(w1, w2)[q]
    v0 = pair(wb2[0], wb2[1], z, wb2[0])
    v1 = pair(wb2[2], wb2[3], wb2[1], wb2[2])
    v2 = pair(wb2[4], z, wb2[3], wb2[4])
    f01 = jnp.concatenate([wf1[0], wf1[1]], axis=0)
    f23 = jnp.concatenate([wf1[2], wf1[3]], axis=0)
    f4 = wf1[4]
    g = np.arange(5 * NB)
    cols = 8 * (g % NB) + g // NB
    s = np.zeros((5 * NB, L2), np.float32)
    s[g, cols] = 1.0
    return g0, g1, h0, h1, v0, v1, v2, jnp.asarray(s, bf16), f01, f23, f4


def kernel(x, wb1, b1, wb2, b2, wf1, bf1, wf2, bf2, wf3, bf3):
    Bt, Cc, H, W = x.shape
    assert (Cc, H, W) == (3, 32, 32)
    nsteps = max(1, -(-Bt // NB))
    Bp = nsteps * NB
    if Bp != Bt:
        x = jnp.pad(x, ((0, Bp - Bt), (0, 0), (0, 0), (0, 0)))
    xv = x.reshape(nsteps, NB, 3, 8, 128)

    g0, g1, h0, h1, v0, v1, v2, s, f01, f23, f4 = _pack_weights(wb1, wb2, wf1)

    c2 = lambda i: (0, 0)
    full = lambda shape: pl.BlockSpec(shape, c2)
    out = pl.pallas_call(
        _lenet_body,
        out_shape=jax.ShapeDtypeStruct((nsteps, NB, 128), jnp.float32),
        grid=(nsteps,),
        in_specs=[
            pl.BlockSpec((1, NB, 3, 8, 128), lambda i: (i, 0, 0, 0, 0)),
            full((384, 512)), full((384, 512)),
            full((384, 512)), full((384, 512)),
            full((1, 128)),
            full((256, 512)), full((256, 512)), full((256, 512)),
            full((1, 128)),
            full((5 * NB, L2)),
            full((256, 128)), full((256, 128)), full((128, 128)),
            full((1, 128)),
            full((128, 128)), full((1, 128)),
            full((128, 128)), full((1, 128)),
        ],
        out_specs=pl.BlockSpec((1, NB, 128), lambda i: (i, 0, 0)),
        compiler_params=pltpu.CompilerParams(
            dimension_semantics=("parallel",),
            vmem_limit_bytes=64 * 1024 * 1024),
    )(xv, g0, g1, h0, h1, b1, v0, v1, v2, b2, s,
      f01, f23, f4, bf1, wf2, bf2, wf3, bf3)
    return out.reshape(Bp, 128)[:Bt, :10]


# speedup vs baseline: 3.2276x; 1.0566x over previous
import functools

import jax
import jax.numpy as jnp
import numpy as np
from jax.experimental import pallas as pl
from jax.experimental.pallas import tpu as pltpu

NB = 128
L = NB * 8
L2 = L - 2
CH = 32
SC = 5 * CH

_ROWS = dict(g0=(0, 384), g1=(384, 384), h0=(768, 384), h1=(1152, 384),
             v0=(1536, 256), v1=(1792, 256), v2=(2048, 256),
             f01=(2304, 256), f23=(2560, 256), f4=(2816, 128),
             wf2=(2944, 128), wf3=(3072, 128))
_WROWS = 3200


def _lenet_body(x_ref, wc_hbm, s_hbm, bias_hbm, o_ref,
                wc, s, bias, sems):
    f32, bf16 = jnp.float32, jnp.bfloat16
    dot = functools.partial(jnp.dot, preferred_element_type=f32)

    @pl.when(pl.program_id(0) == 0)
    def _load_weights():
        pltpu.make_async_copy(wc_hbm, wc, sems.at[0]).start()
        pltpu.make_async_copy(s_hbm, s, sems.at[1]).start()
        pltpu.make_async_copy(bias_hbm, bias, sems.at[2]).start()
        pltpu.make_async_copy(wc_hbm, wc, sems.at[0]).wait()
        pltpu.make_async_copy(s_hbm, s, sems.at[1]).wait()
        pltpu.make_async_copy(bias_hbm, bias, sems.at[2]).wait()

    def W(name, lanes=512):
        r0, nr = _ROWS[name]
        return wc[r0:r0 + nr, :lanes]

    xb = x_ref[0]
    X = jnp.concatenate(
        [xb[:, c].reshape(NB * 8, 128).astype(bf16) for c in range(3)],
        axis=1)
    X = jnp.concatenate([X, jnp.zeros((8, 384), bf16)], axis=0)

    def pool_relu(y, brow):
        m = jnp.maximum(y[:, :256], y[:, 256:])
        m = jnp.maximum(m[:, :128], m[:, 128:]) + bias[brow:brow + 1]
        return jnp.maximum(m, 0.0).astype(bf16)

    y01 = dot(X[0:L], W("g0")) + dot(X[1:1 + L], W("g1"))
    ze = pool_relu(y01, 0)
    y23 = dot(X[0:L], W("h0")) + dot(X[1:1 + L], W("h1"))
    zo = pool_relu(y23, 0)
    C = jnp.concatenate([ze, zo], axis=1)

    u = (dot(C[0:L2], W("v0")) + dot(C[1:1 + L2], W("v1"))
         + dot(C[2:2 + L2], W("v2")))
    p2 = pool_relu(u, 1)

    svals = s[...]
    h1s = []
    for c in range(NB // CH):
        p2c = p2[c * CH * 8:c * CH * 8 + SC * 2 - 66]
        sg = dot(svals, p2c).astype(bf16)
        h1 = (dot(jnp.concatenate([sg[0:CH], sg[CH:2 * CH]], axis=1),
                  W("f01", 128))
              + dot(jnp.concatenate([sg[2 * CH:3 * CH], sg[3 * CH:4 * CH]],
                                    axis=1), W("f23", 128))
              + dot(sg[4 * CH:5 * CH], W("f4", 128)))
        h1s.append(jnp.maximum(h1 + bias[2:3], 0.0).astype(bf16))
    h1 = jnp.concatenate(h1s, axis=0)
    h2 = jnp.maximum(dot(h1, W("wf2", 128)) + bias[3:4], 0.0).astype(bf16)
    o_ref[0] = dot(h2, W("wf3", 128)) + bias[4:5]


def _pack_weights(wb1, wb2, wf1):
    bf16 = jnp.bfloat16
    z = jnp.zeros((128, 256), bf16)

    def pair(wa, wb, wc_, wd):
        return jnp.concatenate(
            [jnp.concatenate([wa, wb], axis=0),
             jnp.concatenate([wc_, wd], axis=0)], axis=1)

    w0 = pair(wb1[0], wb1[1], z, wb1[0])
    w1 = pair(wb1[2], wb1[3], wb1[1], wb1[2])
    w2 = pair(wb1[4], z, wb1[3], wb1[4])
    c, r4, w = np.meshgrid(np.arange(3), np.arange(4), np.arange(32),
                           indexing="ij")
    q = jnp.asarray((r4 * 128 + w * 4 + c).reshape(384))
    z512 = jnp.zeros((256, 512), bf16)
    vs = lambda a, b: jnp.concatenate([a, b], axis=0)
    parts = {
        "g0": vs(w0, w1)[q], "g1": vs(w2, z512)[q],
        "h0": vs(z512, w0)[q], "h1": vs(w1, w2)[q],
        "v0": pair(wb2[0], wb2[1], z, wb2[0]),
        "v1": pair(wb2[2], wb2[3], wb2[1], wb2[2]),
        "v2": pair(wb2[4], z, wb2[3], wb2[4]),
        "f01": jnp.concatenate([wf1[0], wf1[1]], axis=0),
        "f23": jnp.concatenate([wf1[2], wf1[3]], axis=0),
        "f4": wf1[4],
    }
    wc = jnp.zeros((_WROWS, 512), bf16)
    for name, (r0, nr) in _ROWS.items():
        if name in parts:
            p = parts[name]
            wc = wc.at[r0:r0 + nr, :p.shape[1]].set(p)
    g = np.arange(SC)
    cols = 8 * (g % CH) + g // CH
    s = np.zeros((SC, SC * 2 - 66), np.float32)
    s[g, cols] = 1.0
    return wc, jnp.asarray(s, bf16)


def kernel(x, wb1, b1, wb2, b2, wf1, bf1, wf2, bf2, wf3, bf3):
    Bt, Cc, H, Wd = x.shape
    assert (Cc, H, Wd) == (3, 32, 32)
    nsteps = max(1, -(-Bt // NB))
    Bp = nsteps * NB
    if Bp != Bt:
        x = jnp.pad(x, ((0, Bp - Bt), (0, 0), (0, 0), (0, 0)))
    xv = x.reshape(nsteps, NB, 3, 8, 128)

    wc, s = _pack_weights(wb1, wb2, wf1)
    wc = wc.at[_ROWS["wf2"][0]:_ROWS["wf2"][0] + 128, :128].set(wf2)
    wc = wc.at[_ROWS["wf3"][0]:_ROWS["wf3"][0] + 128, :128].set(wf3)
    bias = jnp.concatenate([b1, b2, bf1, bf2, bf3], axis=0)

    out = pl.pallas_call(
        _lenet_body,
        out_shape=jax.ShapeDtypeStruct((nsteps, NB, 128), jnp.float32),
        grid=(nsteps,),
        in_specs=[
            pl.BlockSpec((1, NB, 3, 8, 128), lambda i: (i, 0, 0, 0, 0)),
            pl.BlockSpec(memory_space=pl.ANY),
            pl.BlockSpec(memory_space=pl.ANY),
            pl.BlockSpec(memory_space=pl.ANY),
        ],
        out_specs=pl.BlockSpec((1, NB, 128), lambda i: (i, 0, 0)),
        scratch_shapes=[
            pltpu.VMEM((_WROWS, 512), jnp.bfloat16),
            pltpu.VMEM((SC, SC * 2 - 66), jnp.bfloat16),
            pltpu.VMEM((5, 128), jnp.float32),
            pltpu.SemaphoreType.DMA((3,)),
        ],
        compiler_params=pltpu.CompilerParams(
            dimension_semantics=("arbitrary",),
            vmem_limit_bytes=64 * 1024 * 1024),
    )(xv, wc, s, bias)
    return out.reshape(Bp, 128)[:Bt, :10]


# speedup vs baseline: 3.2761x; 1.0150x over previous
import functools

import jax
import jax.numpy as jnp
import numpy as np
from jax.experimental import pallas as pl
from jax.experimental.pallas import tpu as pltpu

NB = 256
L = NB * 8
L2 = L - 2
CH = 32
SC = 5 * CH

_ROWS = dict(g0=(0, 384), g1=(384, 384), h0=(768, 384), h1=(1152, 384),
             v0=(1536, 256), v1=(1792, 256), v2=(2048, 256),
             f01=(2304, 256), f23=(2560, 256), f4=(2816, 128),
             wf2=(2944, 128), wf3=(3072, 128))
_WROWS = 3200


def _lenet_body(x_ref, wc_hbm, s_hbm, bias_hbm, o_ref,
                wc, s, bias, sems):
    f32, bf16 = jnp.float32, jnp.bfloat16
    dot = functools.partial(jnp.dot, preferred_element_type=f32)

    @pl.when(pl.program_id(0) == 0)
    def _load_weights():
        pltpu.make_async_copy(wc_hbm, wc, sems.at[0]).start()
        pltpu.make_async_copy(s_hbm, s, sems.at[1]).start()
        pltpu.make_async_copy(bias_hbm, bias, sems.at[2]).start()
        pltpu.make_async_copy(wc_hbm, wc, sems.at[0]).wait()
        pltpu.make_async_copy(s_hbm, s, sems.at[1]).wait()
        pltpu.make_async_copy(bias_hbm, bias, sems.at[2]).wait()

    def W(name, lanes=512):
        r0, nr = _ROWS[name]
        return wc[r0:r0 + nr, :lanes]

    xb = x_ref[0]
    X = jnp.concatenate(
        [xb[:, c].reshape(NB * 8, 128).astype(bf16) for c in range(3)],
        axis=1)
    X = jnp.concatenate([X, jnp.zeros((8, 384), bf16)], axis=0)

    def pool_relu(y, brow):
        m = jnp.maximum(y[:, :256], y[:, 256:])
        m = jnp.maximum(m[:, :128], m[:, 128:]) + bias[brow:brow + 1]
        return jnp.maximum(m, 0.0).astype(bf16)

    y01 = dot(X[0:L], W("g0")) + dot(X[1:1 + L], W("g1"))
    ze = pool_relu(y01, 0)
    y23 = dot(X[0:L], W("h0")) + dot(X[1:1 + L], W("h1"))
    zo = pool_relu(y23, 0)
    C = jnp.concatenate([ze, zo], axis=1)

    u = (dot(C[0:L2], W("v0")) + dot(C[1:1 + L2], W("v1"))
         + dot(C[2:2 + L2], W("v2")))
    p2 = pool_relu(u, 1)

    svals = s[...]
    h1s = []
    for c in range(NB // CH):
        p2c = p2[c * CH * 8:c * CH * 8 + SC * 2 - 66]
        sg = dot(svals, p2c).astype(bf16)
        h1 = (dot(jnp.concatenate([sg[0:CH], sg[CH:2 * CH]], axis=1),
                  W("f01", 128))
              + dot(jnp.concatenate([sg[2 * CH:3 * CH], sg[3 * CH:4 * CH]],
                                    axis=1), W("f23", 128))
              + dot(sg[4 * CH:5 * CH], W("f4", 128)))
        h1s.append(jnp.maximum(h1 + bias[2:3], 0.0).astype(bf16))
    h1 = jnp.concatenate(h1s, axis=0)
    h2 = jnp.maximum(dot(h1, W("wf2", 128)) + bias[3:4], 0.0).astype(bf16)
    o_ref[0] = dot(h2, W("wf3", 128)) + bias[4:5]


def _pack_weights(wb1, wb2, wf1):
    bf16 = jnp.bfloat16
    z = jnp.zeros((128, 256), bf16)

    def pair(wa, wb, wc_, wd):
        return jnp.concatenate(
            [jnp.concatenate([wa, wb], axis=0),
             jnp.concatenate([wc_, wd], axis=0)], axis=1)

    w0 = pair(wb1[0], wb1[1], z, wb1[0])
    w1 = pair(wb1[2], wb1[3], wb1[1], wb1[2])
    w2 = pair(wb1[4], z, wb1[3], wb1[4])
    c, r4, w = np.meshgrid(np.arange(3), np.arange(4), np.arange(32),
                           indexing="ij")
    q = jnp.asarray((r4 * 128 + w * 4 + c).reshape(384))
    z512 = jnp.zeros((256, 512), bf16)
    vs = lambda a, b: jnp.concatenate([a, b], axis=0)
    parts = {
        "g0": vs(w0, w1)[q], "g1": vs(w2, z512)[q],
        "h0": vs(z512, w0)[q], "h1": vs(w1, w2)[q],
        "v0": pair(wb2[0], wb2[1], z, wb2[0]),
        "v1": pair(wb2[2], wb2[3], wb2[1], wb2[2]),
        "v2": pair(wb2[4], z, wb2[3], wb2[4]),
        "f01": jnp.concatenate([wf1[0], wf1[1]], axis=0),
        "f23": jnp.concatenate([wf1[2], wf1[3]], axis=0),
        "f4": wf1[4],
    }
    wc = jnp.zeros((_WROWS, 512), bf16)
    for name, (r0, nr) in _ROWS.items():
        if name in parts:
            p = parts[name]
            wc = wc.at[r0:r0 + nr, :p.shape[1]].set(p)
    g = np.arange(SC)
    cols = 8 * (g % CH) + g // CH
    s = np.zeros((SC, SC * 2 - 66), np.float32)
    s[g, cols] = 1.0
    return wc, jnp.asarray(s, bf16)


def kernel(x, wb1, b1, wb2, b2, wf1, bf1, wf2, bf2, wf3, bf3):
    Bt, Cc, H, Wd = x.shape
    assert (Cc, H, Wd) == (3, 32, 32)
    nsteps = max(1, -(-Bt // NB))
    Bp = nsteps * NB
    if Bp != Bt:
        x = jnp.pad(x, ((0, Bp - Bt), (0, 0), (0, 0), (0, 0)))
    xv = x.reshape(nsteps, NB, 3, 8, 128)

    wc, s = _pack_weights(wb1, wb2, wf1)
    wc = wc.at[_ROWS["wf2"][0]:_ROWS["wf2"][0] + 128, :128].set(wf2)
    wc = wc.at[_ROWS["wf3"][0]:_ROWS["wf3"][0] + 128, :128].set(wf3)
    bias = jnp.concatenate([b1, b2, bf1, bf2, bf3], axis=0)

    out = pl.pallas_call(
        _lenet_body,
        out_shape=jax.ShapeDtypeStruct((nsteps, NB, 128), jnp.float32),
        grid=(nsteps,),
        in_specs=[
            pl.BlockSpec((1, NB, 3, 8, 128), lambda i: (i, 0, 0, 0, 0)),
            pl.BlockSpec(memory_space=pl.ANY),
            pl.BlockSpec(memory_space=pl.ANY),
            pl.BlockSpec(memory_space=pl.ANY),
        ],
        out_specs=pl.BlockSpec((1, NB, 128), lambda i: (i, 0, 0)),
        scratch_shapes=[
            pltpu.VMEM((_WROWS, 512), jnp.bfloat16),
            pltpu.VMEM((SC, SC * 2 - 66), jnp.bfloat16),
            pltpu.VMEM((5, 128), jnp.float32),
            pltpu.SemaphoreType.DMA((3,)),
        ],
        compiler_params=pltpu.CompilerParams(
            dimension_semantics=("arbitrary",),
            vmem_limit_bytes=64 * 1024 * 1024),
    )(xv, wc, s, bias)
    return out.reshape(Bp, 128)[:Bt, :10]
